# baseline (speedup 1.0000x reference)
import functools

import jax
import jax.numpy as jnp
from jax import lax
from jax.experimental import pallas as pl
from jax.experimental.pallas import tpu as pltpu
from jax.experimental.pallas import tpu_sc as plsc

E = 8
K = 2
D = 768
T = 32768
LANES = 16
NUM_CORES = 2
NUM_SUBCORES = 16
NW = NUM_CORES * NUM_SUBCORES
TPW = T // NW
TM = 1024


def _logits_body(h_ref, wt_ref, out_ref):
    out_ref[...] = lax.dot_general(
        h_ref[...], wt_ref[...], (((1,), (0,)), ((), ())),
        preferred_element_type=jnp.float32,
        precision=lax.Precision.DEFAULT,
    )


def _logits_tc(h, wt):
    return pl.pallas_call(
        _logits_body,
        grid=(T // TM,),
        in_specs=[
            pl.BlockSpec((TM, D), lambda i: (i, 0)),
            pl.BlockSpec((D, E), lambda i: (0, 0)),
        ],
        out_specs=pl.BlockSpec((TM, E), lambda i: (i, 0)),
        out_shape=jax.ShapeDtypeStruct((T, E), jnp.float32),
    )(h, wt)


def _route_body(logits_hbm, w_hbm, i_hbm, lg_v, w_v, i_v, sem):
    wid = lax.axis_index("c") * NUM_SUBCORES + lax.axis_index("s")
    base = wid * TPW
    pltpu.async_copy(logits_hbm.at[pl.ds(base * E, TPW * E)], lg_v, sem).wait()

    iota = lax.broadcasted_iota(jnp.int32, (LANES,), 0)
    zeros = jnp.zeros((LANES,), jnp.int32)
    ones = jnp.full((LANES,), 1, jnp.int32)

    @pl.loop(0, TPW, step=LANES)
    def _chunk(t0):
        row8 = (t0 + iota) * E
        l0 = plsc.load_gather(lg_v, [row8])
        l1 = plsc.load_gather(lg_v, [row8 + 1])
        gt = l1 > l0
        best = jnp.where(gt, l1, l0)
        bidx = jnp.where(gt, ones, zeros)
        second = jnp.where(gt, l0, l1)
        sidx = jnp.where(gt, zeros, ones)
        for e in range(2, E):
            ev = jnp.full((LANES,), e, jnp.int32)
            le = plsc.load_gather(lg_v, [row8 + e])
            gt_b = le > best
            gt_s = le > second
            second = jnp.where(gt_b, best, jnp.where(gt_s, le, second))
            sidx = jnp.where(gt_b, bidx, jnp.where(gt_s, ev, sidx))
            best = jnp.where(gt_b, le, best)
            bidx = jnp.where(gt_b, ev, bidx)
        r = jnp.exp(second - best)
        denom = r + 1.0
        w1 = 1.0 / denom
        w2 = r / denom
        row2 = (t0 + iota) * K
        plsc.store_scatter(w_v, [row2], w1)
        plsc.store_scatter(w_v, [row2 + 1], w2)
        plsc.store_scatter(i_v, [row2], bidx)
        plsc.store_scatter(i_v, [row2 + 1], sidx)

    pltpu.async_copy(w_v, w_hbm.at[pl.ds(base * K, TPW * K)], sem).wait()
    pltpu.async_copy(i_v, i_hbm.at[pl.ds(base * K, TPW * K)], sem).wait()


@functools.cache
def _route_sc():
    return pl.kernel(
        _route_body,
        out_type=[
            jax.ShapeDtypeStruct((T * K,), jnp.float32),
            jax.ShapeDtypeStruct((T * K,), jnp.int32),
        ],
        mesh=plsc.VectorSubcoreMesh(
            core_axis_name="c", subcore_axis_name="s",
            num_cores=NUM_CORES, num_subcores=NUM_SUBCORES,
        ),
        scratch_types=[
            pltpu.VMEM((TPW * E,), jnp.float32),
            pltpu.VMEM((TPW * K,), jnp.float32),
            pltpu.VMEM((TPW * K,), jnp.int32),
            pltpu.SemaphoreType.DMA,
        ],
        compiler_params=pltpu.CompilerParams(needs_layout_passes=False),
    )


def kernel(hidden_states, W):
    logits = _logits_tc(hidden_states, W.T)
    w_flat, i_flat = _route_sc()(logits.reshape(T * E))
    return logits, w_flat.reshape(T, K), i_flat.reshape(T, K)


# speedup vs baseline: 2.1037x; 2.1037x over previous
import functools

import jax
import jax.numpy as jnp
from jax import lax
from jax.experimental import pallas as pl
from jax.experimental.pallas import tpu as pltpu
from jax.experimental.pallas import tpu_sc as plsc

E = 8
K = 2
D = 768
T = 32768
LANES = 16
NUM_CORES = 2
NUM_SUBCORES = 16
NW = NUM_CORES * NUM_SUBCORES
TPW = T // NW
TM = 1024


def _logits_body(w_ref, h_ref, out_ref):
    out_ref[...] = lax.dot_general(
        w_ref[...], h_ref[...], (((1,), (1,)), ((), ())),
        preferred_element_type=jnp.float32,
        precision=lax.Precision.DEFAULT,
    )


def _logits_tc(h, W):
    return pl.pallas_call(
        _logits_body,
        grid=(T // TM,),
        in_specs=[
            pl.BlockSpec((E, D), lambda i: (0, 0)),
            pl.BlockSpec((TM, D), lambda i: (i, 0)),
        ],
        out_specs=pl.BlockSpec((E, TM), lambda i: (0, i)),
        out_shape=jax.ShapeDtypeStruct((E, T), jnp.float32),
    )(W, h)


def _route_body(logits_hbm, w_hbm, i_hbm, lg_v, w_v, i_v, sem):
    wid = lax.axis_index("c") * NUM_SUBCORES + lax.axis_index("s")
    base = wid * TPW
    copies = [
        pltpu.async_copy(
            logits_hbm.at[pl.ds(e * T + base, TPW)],
            lg_v.at[pl.ds(e * TPW, TPW)], sem)
        for e in range(E)
    ]
    for c in copies:
        c.wait()

    iota = lax.broadcasted_iota(jnp.int32, (LANES,), 0)
    zeros = jnp.zeros((LANES,), jnp.int32)
    ones = jnp.full((LANES,), 1, jnp.int32)

    @pl.loop(0, TPW, step=LANES)
    def _chunk(t0):
        l0 = lg_v[pl.ds(t0, LANES)]
        l1 = lg_v[pl.ds(TPW + t0, LANES)]
        gt = l1 > l0
        best = jnp.where(gt, l1, l0)
        bidx = jnp.where(gt, ones, zeros)
        second = jnp.where(gt, l0, l1)
        sidx = jnp.where(gt, zeros, ones)
        for e in range(2, E):
            ev = jnp.full((LANES,), e, jnp.int32)
            le = lg_v[pl.ds(e * TPW + t0, LANES)]
            gt_b = le > best
            gt_s = le > second
            second = jnp.where(gt_b, best, jnp.where(gt_s, le, second))
            sidx = jnp.where(gt_b, bidx, jnp.where(gt_s, ev, sidx))
            best = jnp.where(gt_b, le, best)
            bidx = jnp.where(gt_b, ev, bidx)
        r = jnp.exp(second - best)
        denom = r + 1.0
        w_v[pl.ds(t0, LANES)] = 1.0 / denom
        w_v[pl.ds(TPW + t0, LANES)] = r / denom
        i_v[pl.ds(t0, LANES)] = bidx
        i_v[pl.ds(TPW + t0, LANES)] = sidx

    outs = [
        pltpu.async_copy(w_v.at[pl.ds(0, TPW)],
                         w_hbm.at[pl.ds(base, TPW)], sem),
        pltpu.async_copy(w_v.at[pl.ds(TPW, TPW)],
                         w_hbm.at[pl.ds(T + base, TPW)], sem),
        pltpu.async_copy(i_v.at[pl.ds(0, TPW)],
                         i_hbm.at[pl.ds(base, TPW)], sem),
        pltpu.async_copy(i_v.at[pl.ds(TPW, TPW)],
                         i_hbm.at[pl.ds(T + base, TPW)], sem),
    ]
    for c in outs:
        c.wait()


@functools.cache
def _route_sc():
    return pl.kernel(
        _route_body,
        out_type=[
            jax.ShapeDtypeStruct((K * T,), jnp.float32),
            jax.ShapeDtypeStruct((K * T,), jnp.int32),
        ],
        mesh=plsc.VectorSubcoreMesh(
            core_axis_name="c", subcore_axis_name="s",
            num_cores=NUM_CORES, num_subcores=NUM_SUBCORES,
        ),
        scratch_types=[
            pltpu.VMEM((E * TPW,), jnp.float32),
            pltpu.VMEM((K * TPW,), jnp.float32),
            pltpu.VMEM((K * TPW,), jnp.int32),
            pltpu.SemaphoreType.DMA,
        ],
        compiler_params=pltpu.CompilerParams(needs_layout_passes=False),
    )


def kernel(hidden_states, W):
    logits_t = _logits_tc(hidden_states, W)
    w_flat, i_flat = _route_sc()(logits_t.reshape(E * T))
    return (
        logits_t.T,
        w_flat.reshape(K, T).T,
        i_flat.reshape(K, T).T,
    )


# speedup vs baseline: 2.5341x; 1.2046x over previous
import functools

import jax
import jax.numpy as jnp
from jax import lax
from jax.experimental import pallas as pl
from jax.experimental.pallas import tpu as pltpu
from jax.experimental.pallas import tpu_sc as plsc

E = 8
K = 2
D = 768
T = 32768
LANES = 16
NUM_CORES = 2
NUM_SUBCORES = 16
NW = NUM_CORES * NUM_SUBCORES
TPW = T // NW
TM = 4096


def _logits_body(w_ref, h_ref, out_ref, flat_ref):
    t = lax.dot_general(
        w_ref[...], h_ref[...], (((1,), (1,)), ((), ())),
        preferred_element_type=jnp.float32,
        precision=lax.Precision.DEFAULT,
    )
    out_ref[...] = t
    flat_ref[...] = jnp.concatenate(
        [t[:, 128 * a:128 * (a + 1)].reshape(E * 128) for a in range(TM // 128)]
    )


def _logits_tc(h, W):
    return pl.pallas_call(
        _logits_body,
        grid=(T // TM,),
        in_specs=[
            pl.BlockSpec((E, D), lambda i: (0, 0)),
            pl.BlockSpec((TM, D), lambda i: (i, 0)),
        ],
        out_specs=[
            pl.BlockSpec((E, TM), lambda i: (0, i)),
            pl.BlockSpec((TM * E,), lambda i: (i,)),
        ],
        out_shape=[
            jax.ShapeDtypeStruct((E, T), jnp.float32),
            jax.ShapeDtypeStruct((T * E,), jnp.float32),
        ],
    )(W, h)


def _route_body(logits_hbm, w_hbm, i_hbm, lg_v, w_v, i_v, sem):
    wid = lax.axis_index("c") * NUM_SUBCORES + lax.axis_index("s")
    base = wid * TPW
    pltpu.async_copy(
        logits_hbm.at[pl.ds(base * E, TPW * E)], lg_v, sem).wait()

    iota = lax.broadcasted_iota(jnp.int32, (LANES,), 0)
    zeros = jnp.zeros((LANES,), jnp.int32)
    ones = jnp.full((LANES,), 1, jnp.int32)

    @pl.loop(0, TPW, step=LANES)
    def _chunk(t0):
        blk = (t0 // 128) * (128 * E) + (t0 % 128)
        l0 = lg_v[pl.ds(blk, LANES)]
        l1 = lg_v[pl.ds(blk + 128, LANES)]
        gt = l1 > l0
        best = jnp.where(gt, l1, l0)
        bidx = jnp.where(gt, ones, zeros)
        second = jnp.where(gt, l0, l1)
        sidx = jnp.where(gt, zeros, ones)
        for e in range(2, E):
            ev = jnp.full((LANES,), e, jnp.int32)
            le = lg_v[pl.ds(blk + 128 * e, LANES)]
            gt_b = le > best
            gt_s = le > second
            second = jnp.where(gt_b, best, jnp.where(gt_s, le, second))
            sidx = jnp.where(gt_b, bidx, jnp.where(gt_s, ev, sidx))
            best = jnp.where(gt_b, le, best)
            bidx = jnp.where(gt_b, ev, bidx)
        r = jnp.exp(second - best)
        denom = r + 1.0
        w_v[pl.ds(t0, LANES)] = 1.0 / denom
        w_v[pl.ds(TPW + t0, LANES)] = r / denom
        i_v[pl.ds(t0, LANES)] = bidx
        i_v[pl.ds(TPW + t0, LANES)] = sidx

    outs = [
        pltpu.async_copy(w_v.at[pl.ds(0, TPW)],
                         w_hbm.at[pl.ds(base, TPW)], sem),
        pltpu.async_copy(w_v.at[pl.ds(TPW, TPW)],
                         w_hbm.at[pl.ds(T + base, TPW)], sem),
        pltpu.async_copy(i_v.at[pl.ds(0, TPW)],
                         i_hbm.at[pl.ds(base, TPW)], sem),
        pltpu.async_copy(i_v.at[pl.ds(TPW, TPW)],
                         i_hbm.at[pl.ds(T + base, TPW)], sem),
    ]
    for c in outs:
        c.wait()


@functools.cache
def _route_sc():
    return pl.kernel(
        _route_body,
        out_type=[
            jax.ShapeDtypeStruct((K * T,), jnp.float32),
            jax.ShapeDtypeStruct((K * T,), jnp.int32),
        ],
        mesh=plsc.VectorSubcoreMesh(
            core_axis_name="c", subcore_axis_name="s",
            num_cores=NUM_CORES, num_subcores=NUM_SUBCORES,
        ),
        scratch_types=[
            pltpu.VMEM((E * TPW,), jnp.float32),
            pltpu.VMEM((K * TPW,), jnp.float32),
            pltpu.VMEM((K * TPW,), jnp.int32),
            pltpu.SemaphoreType.DMA,
        ],
        compiler_params=pltpu.CompilerParams(needs_layout_passes=False),
    )


def kernel(hidden_states, W):
    logits_t, lg_flat = _logits_tc(hidden_states, W)
    w_flat, i_flat = _route_sc()(lg_flat)
    return (
        logits_t.T,
        w_flat.reshape(K, T).T,
        i_flat.reshape(K, T).T,
    )
